# baseline (speedup 1.0000x reference)
import functools

import jax
import jax.numpy as jnp
from jax import lax
from jax.experimental import pallas as pl
from jax.experimental.pallas import tpu as pltpu
from jax.experimental.pallas import tpu_sc as plsc

N = 10000
E = 320000
D_IN = 128
H = 8
DH = 16
HID = H * DH

BLK = 1000


def _k1(x_ref, w_ref, a_ref, feat_ref, elr_ref):
    f = jnp.dot(x_ref[...], w_ref[...], preferred_element_type=jnp.float32)
    feat_ref[...] = f
    elr_ref[...] = jnp.dot(f, a_ref[...], preferred_element_type=jnp.float32)


def _stage1(x, W1, AlAr1):
    return pl.pallas_call(
        _k1,
        grid=(N // BLK,),
        in_specs=[
            pl.BlockSpec((BLK, D_IN), lambda i: (i, 0)),
            pl.BlockSpec((D_IN, HID), lambda i: (0, 0)),
            pl.BlockSpec((HID, 2 * H), lambda i: (0, 0)),
        ],
        out_specs=[
            pl.BlockSpec((BLK, HID), lambda i: (i, 0)),
            pl.BlockSpec((BLK, 2 * H), lambda i: (i, 0)),
        ],
        out_shape=[
            jax.ShapeDtypeStruct((N, HID), jnp.float32),
            jax.ShapeDtypeStruct((N, 2 * H), jnp.float32),
        ],
    )(x, W1, AlAr1)


def _agg_from_table(o0, o1, exp_mat):
    den = jnp.concatenate([o0[:, 64:68], o1[:, 64:68]], axis=1)
    rec = 1.0 / den
    rec_exp = jnp.dot(rec, exp_mat, preferred_element_type=jnp.float32)
    return jnp.concatenate([o0[:, :64], o1[:, :64]], axis=1) * rec_exp


def _k2(o0_ref, o1_ref, b_ref, w_ref, a_ref, exp_ref,
        h1_ref, feat_ref, elr_ref):
    agg = _agg_from_table(o0_ref[...], o1_ref[...], exp_ref[...])
    h1 = jnp.maximum(agg + b_ref[...], 0.0)
    h1_ref[...] = h1
    f = jnp.dot(h1, w_ref[...], preferred_element_type=jnp.float32)
    feat_ref[...] = f
    elr_ref[...] = jnp.dot(f, a_ref[...], preferred_element_type=jnp.float32)


def _stage2(outx_tbl, b1, W2, AlAr2, EXPAND):
    nb = N // BLK
    return pl.pallas_call(
        _k2,
        grid=(nb,),
        in_specs=[
            pl.BlockSpec((BLK, 80), lambda i: (i, 0)),
            pl.BlockSpec((BLK, 80), lambda i, _nb=nb: (_nb + i, 0)),
            pl.BlockSpec((1, HID), lambda i: (0, 0)),
            pl.BlockSpec((HID, HID), lambda i: (0, 0)),
            pl.BlockSpec((HID, 2 * H), lambda i: (0, 0)),
            pl.BlockSpec((H, HID), lambda i: (0, 0)),
        ],
        out_specs=[
            pl.BlockSpec((BLK, HID), lambda i: (i, 0)),
            pl.BlockSpec((BLK, HID), lambda i: (i, 0)),
            pl.BlockSpec((BLK, 2 * H), lambda i: (i, 0)),
        ],
        out_shape=[
            jax.ShapeDtypeStruct((N, HID), jnp.float32),
            jax.ShapeDtypeStruct((N, HID), jnp.float32),
            jax.ShapeDtypeStruct((N, 2 * H), jnp.float32),
        ],
    )(outx_tbl, outx_tbl, b1.reshape(1, HID), W2, AlAr2, EXPAND)


def _k3(o0_ref, o1_ref, h1_ref, b2_ref, g_ref, lb_ref,
        wc1_ref, bc1_ref, wc2_ref, bc2_ref, exp_ref, y_ref):
    agg = _agg_from_table(o0_ref[...], o1_ref[...], exp_ref[...])
    h2 = agg + b2_ref[...]
    hh = h2 + h1_ref[...]
    mu = jnp.mean(hh, axis=-1, keepdims=True)
    c = hh - mu
    var = jnp.mean(c * c, axis=-1, keepdims=True)
    h = c * jax.lax.rsqrt(var + 1e-5) * g_ref[...] + lb_ref[...]
    o1 = jnp.maximum(
        jnp.dot(h, wc1_ref[...], preferred_element_type=jnp.float32)
        + bc1_ref[...], 0.0)
    y_ref[...] = (jnp.dot(o1, wc2_ref[...], preferred_element_type=jnp.float32)
                  + bc2_ref[...])


def _stage3(outx_tbl, h1, b2, ln_g, ln_b, Wc1, bc1, Wc2, bc2, EXPAND):
    nb = N // BLK
    return pl.pallas_call(
        _k3,
        grid=(nb,),
        in_specs=[
            pl.BlockSpec((BLK, 80), lambda i: (i, 0)),
            pl.BlockSpec((BLK, 80), lambda i, _nb=nb: (_nb + i, 0)),
            pl.BlockSpec((BLK, HID), lambda i: (i, 0)),
            pl.BlockSpec((1, HID), lambda i: (0, 0)),
            pl.BlockSpec((1, HID), lambda i: (0, 0)),
            pl.BlockSpec((1, HID), lambda i: (0, 0)),
            pl.BlockSpec((HID, HID), lambda i: (0, 0)),
            pl.BlockSpec((1, HID), lambda i: (0, 0)),
            pl.BlockSpec((HID, 1), lambda i: (0, 0)),
            pl.BlockSpec((1, 1), lambda i: (0, 0)),
            pl.BlockSpec((H, HID), lambda i: (0, 0)),
        ],
        out_specs=pl.BlockSpec((BLK, 1), lambda i: (i, 0)),
        out_shape=jax.ShapeDtypeStruct((N, 1), jnp.float32),
    )(outx_tbl, outx_tbl, h1, b2.reshape(1, HID),
      ln_g.reshape(1, HID), ln_b.reshape(1, HID), Wc1, bc1.reshape(1, HID),
      Wc2, bc2.reshape(1, 1), EXPAND)


CHUNK = 80
NCHUNK = E // CHUNK
TILES = 16
CPT = NCHUNK // TILES
SLAB = 624
TAIL = N - TILES * SLAB


def _sc_body(featx_hbm, tblB_hbm, src_hbm, dst_hbm, z80_hbm, outx_hbm,
             outx_sh, srcv, dstv, dofs0, dofs1, fx0, fx1, lb0, lb1,
             sem_i, sg0, sg1, ss0, ss1):
    c = lax.axis_index("c")
    s = lax.axis_index("s")
    base = (c * N).astype(jnp.int32)

    r0 = s * SLAB
    pltpu.sync_copy(z80_hbm.at[pl.ds(r0, SLAB)], outx_sh.at[pl.ds(r0, SLAB)])

    @pl.when(s == TILES - 1)
    def _zero_tail():
        t0 = TILES * SLAB
        pltpu.sync_copy(z80_hbm.at[pl.ds(t0, TAIL)],
                        outx_sh.at[pl.ds(t0, TAIL)])

    plsc.subcore_barrier()

    dofs = (dofs0, dofs1)
    fx = (fx0, fx1)
    lb = (lb0, lb1)
    sg = (sg0, sg1)
    ss = (ss0, ss1)

    def issue_idx(j):
        r = jnp.bitwise_and(j, 3)
        eb = (s + j * TILES) * CHUNK
        pltpu.async_copy(src_hbm.at[pl.ds(eb, CHUNK)], srcv.at[r], sem_i)
        pltpu.async_copy(dst_hbm.at[pl.ds(eb, CHUNK)], dstv.at[r], sem_i)

    def wait_idx_and_offset(j, p):
        r = jnp.bitwise_and(j, 3)
        pltpu.make_async_copy(src_hbm.at[pl.ds(0, CHUNK)], srcv.at[r],
                              sem_i).wait()
        pltpu.make_async_copy(dst_hbm.at[pl.ds(0, CHUNK)], dstv.at[r],
                              sem_i).wait()
        for k in range(CHUNK // 16):
            sl = pl.ds(k * 16, 16)
            srcv[r, sl] = srcv[r, sl] + base
            dofs[p][sl] = dstv[r, sl] + base

    def issue_gather(j, p):
        r = jnp.bitwise_and(j, 3)
        pltpu.async_copy(featx_hbm.at[srcv.at[r]], fx[p], sg[p])
        pltpu.async_copy(tblB_hbm.at[dofs[p]], lb[p], sg[p])

    def wait_gather(p):
        pltpu.make_async_copy(featx_hbm.at[srcv.at[0]], fx[p], sg[p]).wait()
        pltpu.make_async_copy(tblB_hbm.at[dofs[p]], lb[p], sg[p]).wait()

    def compute(p):
        fxp = fx[p]
        lbp = lb[p]

        def edge(i, _):
            a = fxp[i, pl.ds(64, 16)]
            e = a + lbp[i]
            e = jnp.maximum(e, 0.2 * e)
            ee = jnp.exp(e)
            fxp[i, pl.ds(64, 16)] = ee
            for h in range(4):
                hidx = jnp.full((16,), h, jnp.int32)
                sp = ee.at[hidx].get(mode="promise_in_bounds")
                csl = pl.ds(h * 16, 16)
                fxp[i, csl] = fxp[i, csl] * sp
            return 0

        lax.fori_loop(0, CHUNK, edge, 0, unroll=8)

    def issue_scatter(j, p):
        r = jnp.bitwise_and(j, 3)
        pltpu.async_copy(fx[p], outx_sh.at[dstv.at[r]], ss[p], add=True)

    def wait_scatter(p):
        pltpu.make_async_copy(fx[p], outx_sh.at[dstv.at[0]], ss[p]).wait()

    issue_idx(jnp.int32(0))
    wait_idx_and_offset(jnp.int32(0), 0)
    issue_gather(jnp.int32(0), 0)
    issue_idx(jnp.int32(1))

    def pair_body(j2, _):
        for u in range(2):
            j = 2 * j2 + u
            p = u
            q = 1 - u

            @pl.when(j >= 1)
            def _w():
                wait_scatter(q)

            @pl.when(j <= CPT - 2)
            def _og():
                wait_idx_and_offset(j + 1, q)
                issue_gather(j + 1, q)

            wait_gather(p)
            compute(p)
            issue_scatter(j, p)

            @pl.when(j <= CPT - 3)
            def _i():
                issue_idx(j + 2)
        return 0

    lax.fori_loop(0, CPT // 2, pair_body, 0)
    wait_scatter(1)
    plsc.subcore_barrier()

    o0 = c * N + r0
    pltpu.sync_copy(outx_sh.at[pl.ds(r0, SLAB)], outx_hbm.at[pl.ds(o0, SLAB)])

    @pl.when(s == TILES - 1)
    def _write_tail():
        t0 = TILES * SLAB
        ot = c * N + t0
        pltpu.sync_copy(outx_sh.at[pl.ds(t0, TAIL)],
                        outx_hbm.at[pl.ds(ot, TAIL)])


def _edge_phase_sc(featx_tbl, tblB, src, dst):
    mesh = plsc.VectorSubcoreMesh(core_axis_name="c", subcore_axis_name="s")
    f = pl.kernel(
        _sc_body,
        compiler_params=pltpu.CompilerParams(use_tc_tiling_on_sc=False),
        out_type=jax.ShapeDtypeStruct((2 * N, 80), jnp.float32),
        mesh=mesh,
        scratch_types=[
            pltpu.VMEM_SHARED((N, 80), jnp.float32),
            pltpu.VMEM((4, CHUNK), jnp.int32),
            pltpu.VMEM((4, CHUNK), jnp.int32),
            pltpu.VMEM((CHUNK,), jnp.int32),
            pltpu.VMEM((CHUNK,), jnp.int32),
            pltpu.VMEM((CHUNK, 80), jnp.float32),
            pltpu.VMEM((CHUNK, 80), jnp.float32),
            pltpu.VMEM((CHUNK, 16), jnp.float32),
            pltpu.VMEM((CHUNK, 16), jnp.float32),
            pltpu.SemaphoreType.DMA,
            pltpu.SemaphoreType.DMA,
            pltpu.SemaphoreType.DMA,
            pltpu.SemaphoreType.DMA,
            pltpu.SemaphoreType.DMA,
        ],
    )
    z80 = jnp.zeros((N, 80), jnp.float32)
    return f(featx_tbl, tblB, src, dst, z80)


def _build_alar(al, ar):
    idx = jnp.arange(HID)
    head = idx // DH
    A = jnp.zeros((HID, 2 * H), jnp.float32)
    A = A.at[idx, head].set(al.reshape(-1))
    A = A.at[idx, H + head].set(ar.reshape(-1))
    return A


def _split_tables(feat, elr):
    el = elr[:, :H]
    er = elr[:, H:]
    fx0 = jnp.concatenate([feat[:, :64], jnp.tile(el[:, :4], (1, 4))], axis=1)
    fx1 = jnp.concatenate([feat[:, 64:], jnp.tile(el[:, 4:], (1, 4))], axis=1)
    featx = jnp.concatenate([fx0, fx1], axis=0)
    tblB = jnp.concatenate([jnp.tile(er[:, :4], (1, 4)),
                            jnp.tile(er[:, 4:], (1, 4))], axis=0)
    return featx, tblB


_EXPAND = None


def _expand_mat():
    idx = jnp.arange(HID)
    return (jnp.arange(H)[:, None] == (idx // DH)[None, :]).astype(jnp.float32)


def kernel(x, edge_index, W1, al1, ar1, b1, W2, al2, ar2, b2, ln_g, ln_b,
           Wc1, bc1, Wc2, bc2):
    src = edge_index[0]
    dst = edge_index[1]
    EXPAND = _expand_mat()

    feat1, elr1 = _stage1(x, W1, _build_alar(al1, ar1))
    ft1, tB1 = _split_tables(feat1, elr1)
    outx1 = _edge_phase_sc(ft1, tB1, src, dst)
    h1, feat2, elr2 = _stage2(outx1, b1, W2, _build_alar(al2, ar2), EXPAND)
    ft2, tB2 = _split_tables(feat2, elr2)
    outx2 = _edge_phase_sc(ft2, tB2, src, dst)
    return _stage3(outx2, h1, b2, ln_g, ln_b, Wc1, bc1, Wc2, bc2, EXPAND)


# speedup vs baseline: 1.6020x; 1.6020x over previous
import functools

import jax
import jax.numpy as jnp
from jax import lax
from jax.experimental import pallas as pl
from jax.experimental.pallas import tpu as pltpu
from jax.experimental.pallas import tpu_sc as plsc

N = 10000
E = 320000
D_IN = 128
H = 8
DH = 16
HID = H * DH

BLK = 1000


def _k1(x_ref, w_ref, a_ref, feat_ref, elr_ref):
    f = jnp.dot(x_ref[...], w_ref[...], preferred_element_type=jnp.float32)
    feat_ref[...] = f
    elr_ref[...] = jnp.dot(f, a_ref[...], preferred_element_type=jnp.float32)


def _stage1(x, W1, AlAr1):
    return pl.pallas_call(
        _k1,
        grid=(N // BLK,),
        in_specs=[
            pl.BlockSpec((BLK, D_IN), lambda i: (i, 0)),
            pl.BlockSpec((D_IN, HID), lambda i: (0, 0)),
            pl.BlockSpec((HID, 2 * H), lambda i: (0, 0)),
        ],
        out_specs=[
            pl.BlockSpec((BLK, HID), lambda i: (i, 0)),
            pl.BlockSpec((BLK, 2 * H), lambda i: (i, 0)),
        ],
        out_shape=[
            jax.ShapeDtypeStruct((N, HID), jnp.float32),
            jax.ShapeDtypeStruct((N, 2 * H), jnp.float32),
        ],
    )(x, W1, AlAr1)


def _agg_from_table(o0, o1, exp_mat):
    den = jnp.concatenate([o0[:, 64:68], o1[:, 64:68]], axis=1)
    rec = 1.0 / den
    rec_exp = jnp.dot(rec, exp_mat, preferred_element_type=jnp.float32)
    return jnp.concatenate([o0[:, :64], o1[:, :64]], axis=1) * rec_exp


def _k2(o0_ref, o1_ref, b_ref, w_ref, a_ref, exp_ref,
        h1_ref, feat_ref, elr_ref):
    agg = _agg_from_table(o0_ref[...], o1_ref[...], exp_ref[...])
    h1 = jnp.maximum(agg + b_ref[...], 0.0)
    h1_ref[...] = h1
    f = jnp.dot(h1, w_ref[...], preferred_element_type=jnp.float32)
    feat_ref[...] = f
    elr_ref[...] = jnp.dot(f, a_ref[...], preferred_element_type=jnp.float32)


def _stage2(outx_tbl, b1, W2, AlAr2, EXPAND):
    nb = N // BLK
    return pl.pallas_call(
        _k2,
        grid=(nb,),
        in_specs=[
            pl.BlockSpec((BLK, 80), lambda i: (i, 0)),
            pl.BlockSpec((BLK, 80), lambda i, _nb=nb: (_nb + i, 0)),
            pl.BlockSpec((1, HID), lambda i: (0, 0)),
            pl.BlockSpec((HID, HID), lambda i: (0, 0)),
            pl.BlockSpec((HID, 2 * H), lambda i: (0, 0)),
            pl.BlockSpec((H, HID), lambda i: (0, 0)),
        ],
        out_specs=[
            pl.BlockSpec((BLK, HID), lambda i: (i, 0)),
            pl.BlockSpec((BLK, HID), lambda i: (i, 0)),
            pl.BlockSpec((BLK, 2 * H), lambda i: (i, 0)),
        ],
        out_shape=[
            jax.ShapeDtypeStruct((N, HID), jnp.float32),
            jax.ShapeDtypeStruct((N, HID), jnp.float32),
            jax.ShapeDtypeStruct((N, 2 * H), jnp.float32),
        ],
    )(outx_tbl, outx_tbl, b1.reshape(1, HID), W2, AlAr2, EXPAND)


def _k3(o0_ref, o1_ref, h1_ref, b2_ref, g_ref, lb_ref,
        wc1_ref, bc1_ref, wc2_ref, bc2_ref, exp_ref, y_ref):
    agg = _agg_from_table(o0_ref[...], o1_ref[...], exp_ref[...])
    h2 = agg + b2_ref[...]
    hh = h2 + h1_ref[...]
    mu = jnp.mean(hh, axis=-1, keepdims=True)
    c = hh - mu
    var = jnp.mean(c * c, axis=-1, keepdims=True)
    h = c * jax.lax.rsqrt(var + 1e-5) * g_ref[...] + lb_ref[...]
    o1 = jnp.maximum(
        jnp.dot(h, wc1_ref[...], preferred_element_type=jnp.float32)
        + bc1_ref[...], 0.0)
    y_ref[...] = (jnp.dot(o1, wc2_ref[...], preferred_element_type=jnp.float32)
                  + bc2_ref[...])


def _stage3(outx_tbl, h1, b2, ln_g, ln_b, Wc1, bc1, Wc2, bc2, EXPAND):
    nb = N // BLK
    return pl.pallas_call(
        _k3,
        grid=(nb,),
        in_specs=[
            pl.BlockSpec((BLK, 80), lambda i: (i, 0)),
            pl.BlockSpec((BLK, 80), lambda i, _nb=nb: (_nb + i, 0)),
            pl.BlockSpec((BLK, HID), lambda i: (i, 0)),
            pl.BlockSpec((1, HID), lambda i: (0, 0)),
            pl.BlockSpec((1, HID), lambda i: (0, 0)),
            pl.BlockSpec((1, HID), lambda i: (0, 0)),
            pl.BlockSpec((HID, HID), lambda i: (0, 0)),
            pl.BlockSpec((1, HID), lambda i: (0, 0)),
            pl.BlockSpec((HID, 1), lambda i: (0, 0)),
            pl.BlockSpec((1, 1), lambda i: (0, 0)),
            pl.BlockSpec((H, HID), lambda i: (0, 0)),
        ],
        out_specs=pl.BlockSpec((BLK, 1), lambda i: (i, 0)),
        out_shape=jax.ShapeDtypeStruct((N, 1), jnp.float32),
    )(outx_tbl, outx_tbl, h1, b2.reshape(1, HID),
      ln_g.reshape(1, HID), ln_b.reshape(1, HID), Wc1, bc1.reshape(1, HID),
      Wc2, bc2.reshape(1, 1), EXPAND)


CHUNK = 80
NCHUNK = E // CHUNK
TILES = 16
CPT = NCHUNK // TILES
SLAB = 624
TAIL = N - TILES * SLAB


def _sc_body(featx_hbm, tblB_hbm, src_hbm, dst_hbm, z80_hbm, outx_hbm,
             outx_sh, srcv, dstv, dofs0, dofs1, fx0, fx1, lb0, lb1,
             sem_i, sg0, sg1, ss0, ss1):
    c = lax.axis_index("c")
    s = lax.axis_index("s")
    base = (c * N).astype(jnp.int32)

    r0 = s * SLAB
    pltpu.sync_copy(z80_hbm.at[pl.ds(r0, SLAB)], outx_sh.at[pl.ds(r0, SLAB)])

    @pl.when(s == TILES - 1)
    def _zero_tail():
        t0 = TILES * SLAB
        pltpu.sync_copy(z80_hbm.at[pl.ds(t0, TAIL)],
                        outx_sh.at[pl.ds(t0, TAIL)])

    plsc.subcore_barrier()

    dofs = (dofs0, dofs1)
    fx = (fx0, fx1)
    lb = (lb0, lb1)
    sg = (sg0, sg1)
    ss = (ss0, ss1)

    def issue_idx(j):
        r = jnp.bitwise_and(j, 3)
        eb = (s + j * TILES) * CHUNK
        pltpu.async_copy(src_hbm.at[pl.ds(eb, CHUNK)], srcv.at[r], sem_i)
        pltpu.async_copy(dst_hbm.at[pl.ds(eb, CHUNK)], dstv.at[r], sem_i)

    def wait_idx_and_offset(j, p):
        r = jnp.bitwise_and(j, 3)
        pltpu.make_async_copy(src_hbm.at[pl.ds(0, CHUNK)], srcv.at[r],
                              sem_i).wait()
        pltpu.make_async_copy(dst_hbm.at[pl.ds(0, CHUNK)], dstv.at[r],
                              sem_i).wait()
        for k in range(CHUNK // 16):
            sl = pl.ds(k * 16, 16)
            srcv[r, sl] = srcv[r, sl] + base
            dofs[p][sl] = dstv[r, sl] + base

    def issue_gather(j, p):
        r = jnp.bitwise_and(j, 3)
        pltpu.async_copy(featx_hbm.at[srcv.at[r]], fx[p], sg[p])
        pltpu.async_copy(tblB_hbm.at[dofs[p]], lb[p], sg[p])

    def wait_gather(p):
        pltpu.make_async_copy(featx_hbm.at[srcv.at[0]], fx[p], sg[p]).wait()
        pltpu.make_async_copy(tblB_hbm.at[dofs[p]], lb[p], sg[p]).wait()

    def compute(p):
        fxp = fx[p]
        lbp = lb[p]

        @plsc.parallel_loop(0, CHUNK, 1, unroll=4)
        def edge(i):
            a = fxp[i, pl.ds(64, 16)]
            e = a + lbp[i]
            e = jnp.maximum(e, 0.2 * e)
            ee = jnp.exp(e)
            fxp[i, pl.ds(64, 16)] = ee
            for h in range(4):
                hidx = jnp.full((16,), h, jnp.int32)
                sp = ee.at[hidx].get(mode="promise_in_bounds")
                csl = pl.ds(h * 16, 16)
                fxp[i, csl] = fxp[i, csl] * sp

    def issue_scatter(j, p):
        r = jnp.bitwise_and(j, 3)
        pltpu.async_copy(fx[p], outx_sh.at[dstv.at[r]], ss[p], add=True)

    def wait_scatter(p):
        pltpu.make_async_copy(fx[p], outx_sh.at[dstv.at[0]], ss[p]).wait()

    issue_idx(jnp.int32(0))
    wait_idx_and_offset(jnp.int32(0), 0)
    issue_gather(jnp.int32(0), 0)
    issue_idx(jnp.int32(1))

    def pair_body(j2, _):
        for u in range(2):
            j = 2 * j2 + u
            p = u
            q = 1 - u

            @pl.when(j >= 1)
            def _w():
                wait_scatter(q)

            @pl.when(j <= CPT - 2)
            def _og():
                wait_idx_and_offset(j + 1, q)
                issue_gather(j + 1, q)

            wait_gather(p)
            compute(p)
            issue_scatter(j, p)

            @pl.when(j <= CPT - 3)
            def _i():
                issue_idx(j + 2)
        return 0

    lax.fori_loop(0, CPT // 2, pair_body, 0)
    wait_scatter(1)
    plsc.subcore_barrier()

    o0 = c * N + r0
    pltpu.sync_copy(outx_sh.at[pl.ds(r0, SLAB)], outx_hbm.at[pl.ds(o0, SLAB)])

    @pl.when(s == TILES - 1)
    def _write_tail():
        t0 = TILES * SLAB
        ot = c * N + t0
        pltpu.sync_copy(outx_sh.at[pl.ds(t0, TAIL)],
                        outx_hbm.at[pl.ds(ot, TAIL)])


def _edge_phase_sc(featx_tbl, tblB, src, dst):
    mesh = plsc.VectorSubcoreMesh(core_axis_name="c", subcore_axis_name="s")
    f = pl.kernel(
        _sc_body,
        compiler_params=pltpu.CompilerParams(use_tc_tiling_on_sc=False),
        out_type=jax.ShapeDtypeStruct((2 * N, 80), jnp.float32),
        mesh=mesh,
        scratch_types=[
            pltpu.VMEM_SHARED((N, 80), jnp.float32),
            pltpu.VMEM((4, CHUNK), jnp.int32),
            pltpu.VMEM((4, CHUNK), jnp.int32),
            pltpu.VMEM((CHUNK,), jnp.int32),
            pltpu.VMEM((CHUNK,), jnp.int32),
            pltpu.VMEM((CHUNK, 80), jnp.float32),
            pltpu.VMEM((CHUNK, 80), jnp.float32),
            pltpu.VMEM((CHUNK, 16), jnp.float32),
            pltpu.VMEM((CHUNK, 16), jnp.float32),
            pltpu.SemaphoreType.DMA,
            pltpu.SemaphoreType.DMA,
            pltpu.SemaphoreType.DMA,
            pltpu.SemaphoreType.DMA,
            pltpu.SemaphoreType.DMA,
        ],
    )
    z80 = jnp.zeros((N, 80), jnp.float32)
    return f(featx_tbl, tblB, src, dst, z80)


def _build_alar(al, ar):
    idx = jnp.arange(HID)
    head = idx // DH
    A = jnp.zeros((HID, 2 * H), jnp.float32)
    A = A.at[idx, head].set(al.reshape(-1))
    A = A.at[idx, H + head].set(ar.reshape(-1))
    return A


def _split_tables(feat, elr):
    el = elr[:, :H]
    er = elr[:, H:]
    fx0 = jnp.concatenate([feat[:, :64], jnp.tile(el[:, :4], (1, 4))], axis=1)
    fx1 = jnp.concatenate([feat[:, 64:], jnp.tile(el[:, 4:], (1, 4))], axis=1)
    featx = jnp.concatenate([fx0, fx1], axis=0)
    tblB = jnp.concatenate([jnp.tile(er[:, :4], (1, 4)),
                            jnp.tile(er[:, 4:], (1, 4))], axis=0)
    return featx, tblB


_EXPAND = None


def _expand_mat():
    idx = jnp.arange(HID)
    return (jnp.arange(H)[:, None] == (idx // DH)[None, :]).astype(jnp.float32)


def kernel(x, edge_index, W1, al1, ar1, b1, W2, al2, ar2, b2, ln_g, ln_b,
           Wc1, bc1, Wc2, bc2):
    src = edge_index[0]
    dst = edge_index[1]
    EXPAND = _expand_mat()

    feat1, elr1 = _stage1(x, W1, _build_alar(al1, ar1))
    ft1, tB1 = _split_tables(feat1, elr1)
    outx1 = _edge_phase_sc(ft1, tB1, src, dst)
    h1, feat2, elr2 = _stage2(outx1, b1, W2, _build_alar(al2, ar2), EXPAND)
    ft2, tB2 = _split_tables(feat2, elr2)
    outx2 = _edge_phase_sc(ft2, tB2, src, dst)
    return _stage3(outx2, h1, b2, ln_g, ln_b, Wc1, bc1, Wc2, bc2, EXPAND)


# speedup vs baseline: 1.8202x; 1.1362x over previous
import functools

import jax
import jax.numpy as jnp
from jax import lax
from jax.experimental import pallas as pl
from jax.experimental.pallas import tpu as pltpu
from jax.experimental.pallas import tpu_sc as plsc

N = 10000
E = 320000
D_IN = 128
H = 8
DH = 16
HID = H * DH

BLK = 1000


def _tables_from(f, elr, csel):
    f64 = jnp.where(csel, f[:, :64], f[:, 64:])
    el4 = jnp.where(csel, elr[:, 0:4], elr[:, 4:8])
    er4 = jnp.where(csel, elr[:, 8:12], elr[:, 12:16])
    fx = jnp.concatenate([f64, jnp.tile(el4, (1, 4))], axis=1)
    return fx, jnp.tile(er4, (1, 4))


def _k1(x_ref, w_ref, a_ref, fx_ref, tb_ref):
    csel = pl.program_id(1) == 0
    f = jnp.dot(x_ref[...], w_ref[...], preferred_element_type=jnp.float32)
    elr = jnp.dot(f, a_ref[...], preferred_element_type=jnp.float32)
    fx, tb = _tables_from(f, elr, csel)
    fx_ref[...] = fx
    tb_ref[...] = tb


def _stage1(x, W1, AlAr1):
    nb = N // BLK
    return pl.pallas_call(
        _k1,
        grid=(nb, 2),
        in_specs=[
            pl.BlockSpec((BLK, D_IN), lambda i, c: (i, 0)),
            pl.BlockSpec((D_IN, HID), lambda i, c: (0, 0)),
            pl.BlockSpec((HID, 2 * H), lambda i, c: (0, 0)),
        ],
        out_specs=[
            pl.BlockSpec((BLK, 80), lambda i, c, _nb=nb: (c * _nb + i, 0)),
            pl.BlockSpec((BLK, 16), lambda i, c, _nb=nb: (c * _nb + i, 0)),
        ],
        out_shape=[
            jax.ShapeDtypeStruct((2 * N, 80), jnp.float32),
            jax.ShapeDtypeStruct((2 * N, 16), jnp.float32),
        ],
    )(x, W1, AlAr1)


def _agg_from_table(o0, o1, exp_mat):
    den = jnp.concatenate([o0[:, 64:68], o1[:, 64:68]], axis=1)
    rec = 1.0 / den
    rec_exp = jnp.dot(rec, exp_mat, preferred_element_type=jnp.float32)
    return jnp.concatenate([o0[:, :64], o1[:, :64]], axis=1) * rec_exp


def _k2(o0_ref, o1_ref, b_ref, w_ref, a_ref, exp_ref,
        h1_ref, fx_ref, tb_ref):
    csel = pl.program_id(1) == 0
    agg = _agg_from_table(o0_ref[...], o1_ref[...], exp_ref[...])
    h1 = jnp.maximum(agg + b_ref[...], 0.0)
    h1_ref[...] = h1
    f = jnp.dot(h1, w_ref[...], preferred_element_type=jnp.float32)
    elr = jnp.dot(f, a_ref[...], preferred_element_type=jnp.float32)
    fx, tb = _tables_from(f, elr, csel)
    fx_ref[...] = fx
    tb_ref[...] = tb


def _stage2(outx_tbl, b1, W2, AlAr2, EXPAND):
    nb = N // BLK
    return pl.pallas_call(
        _k2,
        grid=(nb, 2),
        in_specs=[
            pl.BlockSpec((BLK, 80), lambda i, c: (i, 0)),
            pl.BlockSpec((BLK, 80), lambda i, c, _nb=nb: (_nb + i, 0)),
            pl.BlockSpec((1, HID), lambda i, c: (0, 0)),
            pl.BlockSpec((HID, HID), lambda i, c: (0, 0)),
            pl.BlockSpec((HID, 2 * H), lambda i, c: (0, 0)),
            pl.BlockSpec((H, HID), lambda i, c: (0, 0)),
        ],
        out_specs=[
            pl.BlockSpec((BLK, HID), lambda i, c: (i, 0)),
            pl.BlockSpec((BLK, 80), lambda i, c, _nb=nb: (c * _nb + i, 0)),
            pl.BlockSpec((BLK, 16), lambda i, c, _nb=nb: (c * _nb + i, 0)),
        ],
        out_shape=[
            jax.ShapeDtypeStruct((N, HID), jnp.float32),
            jax.ShapeDtypeStruct((2 * N, 80), jnp.float32),
            jax.ShapeDtypeStruct((2 * N, 16), jnp.float32),
        ],
    )(outx_tbl, outx_tbl, b1.reshape(1, HID), W2, AlAr2, EXPAND)


def _k3(o0_ref, o1_ref, h1_ref, b2_ref, g_ref, lb_ref,
        wc1_ref, bc1_ref, wc2_ref, bc2_ref, exp_ref, y_ref):
    agg = _agg_from_table(o0_ref[...], o1_ref[...], exp_ref[...])
    h2 = agg + b2_ref[...]
    hh = h2 + h1_ref[...]
    mu = jnp.mean(hh, axis=-1, keepdims=True)
    c = hh - mu
    var = jnp.mean(c * c, axis=-1, keepdims=True)
    h = c * jax.lax.rsqrt(var + 1e-5) * g_ref[...] + lb_ref[...]
    o1 = jnp.maximum(
        jnp.dot(h, wc1_ref[...], preferred_element_type=jnp.float32)
        + bc1_ref[...], 0.0)
    y_ref[...] = (jnp.dot(o1, wc2_ref[...], preferred_element_type=jnp.float32)
                  + bc2_ref[...])


def _stage3(outx_tbl, h1, b2, ln_g, ln_b, Wc1, bc1, Wc2, bc2, EXPAND):
    nb = N // BLK
    return pl.pallas_call(
        _k3,
        grid=(nb,),
        in_specs=[
            pl.BlockSpec((BLK, 80), lambda i: (i, 0)),
            pl.BlockSpec((BLK, 80), lambda i, _nb=nb: (_nb + i, 0)),
            pl.BlockSpec((BLK, HID), lambda i: (i, 0)),
            pl.BlockSpec((1, HID), lambda i: (0, 0)),
            pl.BlockSpec((1, HID), lambda i: (0, 0)),
            pl.BlockSpec((1, HID), lambda i: (0, 0)),
            pl.BlockSpec((HID, HID), lambda i: (0, 0)),
            pl.BlockSpec((1, HID), lambda i: (0, 0)),
            pl.BlockSpec((HID, 1), lambda i: (0, 0)),
            pl.BlockSpec((1, 1), lambda i: (0, 0)),
            pl.BlockSpec((H, HID), lambda i: (0, 0)),
        ],
        out_specs=pl.BlockSpec((BLK, 1), lambda i: (i, 0)),
        out_shape=jax.ShapeDtypeStruct((N, 1), jnp.float32),
    )(outx_tbl, outx_tbl, h1, b2.reshape(1, HID),
      ln_g.reshape(1, HID), ln_b.reshape(1, HID), Wc1, bc1.reshape(1, HID),
      Wc2, bc2.reshape(1, 1), EXPAND)


CHUNK = 80
NCHUNK = E // CHUNK
TILES = 16
CPT = NCHUNK // TILES
SLAB = 624
TAIL = N - TILES * SLAB


def _sc_body(featx_hbm, tblB_hbm, src_hbm, dst_hbm, z80_hbm, outx_hbm,
             outx_sh, srcv, dstv, dofs0, dofs1, fx0, fx1, lb0, lb1,
             sem_i, sg0, sg1, ss0, ss1):
    c = lax.axis_index("c")
    s = lax.axis_index("s")
    base = (c * N).astype(jnp.int32)

    r0 = s * SLAB
    pltpu.sync_copy(z80_hbm.at[pl.ds(r0, SLAB)], outx_sh.at[pl.ds(r0, SLAB)])

    @pl.when(s == TILES - 1)
    def _zero_tail():
        t0 = TILES * SLAB
        pltpu.sync_copy(z80_hbm.at[pl.ds(t0, TAIL)],
                        outx_sh.at[pl.ds(t0, TAIL)])

    plsc.subcore_barrier()

    dofs = (dofs0, dofs1)
    fx = (fx0, fx1)
    lb = (lb0, lb1)
    sg = (sg0, sg1)
    ss = (ss0, ss1)

    def issue_idx(j):
        r = jnp.bitwise_and(j, 3)
        eb = (s + j * TILES) * CHUNK
        pltpu.async_copy(src_hbm.at[pl.ds(eb, CHUNK)], srcv.at[r], sem_i)
        pltpu.async_copy(dst_hbm.at[pl.ds(eb, CHUNK)], dstv.at[r], sem_i)

    def wait_idx_and_offset(j, p):
        r = jnp.bitwise_and(j, 3)
        pltpu.make_async_copy(src_hbm.at[pl.ds(0, CHUNK)], srcv.at[r],
                              sem_i).wait()
        pltpu.make_async_copy(dst_hbm.at[pl.ds(0, CHUNK)], dstv.at[r],
                              sem_i).wait()
        for k in range(CHUNK // 16):
            sl = pl.ds(k * 16, 16)
            srcv[r, sl] = srcv[r, sl] + base
            dofs[p][sl] = dstv[r, sl] + base

    def issue_gather(j, p):
        r = jnp.bitwise_and(j, 3)
        pltpu.async_copy(featx_hbm.at[srcv.at[r]], fx[p], sg[p])
        pltpu.async_copy(tblB_hbm.at[dofs[p]], lb[p], sg[p])

    def wait_gather(p):
        pltpu.make_async_copy(featx_hbm.at[srcv.at[0]], fx[p], sg[p]).wait()
        pltpu.make_async_copy(tblB_hbm.at[dofs[p]], lb[p], sg[p]).wait()

    def compute(p):
        fxp = fx[p]
        lbp = lb[p]

        @plsc.parallel_loop(0, CHUNK, 1, unroll=4)
        def edge(i):
            a = fxp[i, pl.ds(64, 16)]
            e = a + lbp[i]
            e = jnp.maximum(e, 0.2 * e)
            ee = jnp.exp(e)
            fxp[i, pl.ds(64, 16)] = ee
            for h in range(4):
                hidx = jnp.full((16,), h, jnp.int32)
                sp = ee.at[hidx].get(mode="promise_in_bounds")
                csl = pl.ds(h * 16, 16)
                fxp[i, csl] = fxp[i, csl] * sp

    def issue_scatter(j, p):
        r = jnp.bitwise_and(j, 3)
        pltpu.async_copy(fx[p], outx_sh.at[dstv.at[r]], ss[p], add=True)

    def wait_scatter(p):
        pltpu.make_async_copy(fx[p], outx_sh.at[dstv.at[0]], ss[p]).wait()

    issue_idx(jnp.int32(0))
    wait_idx_and_offset(jnp.int32(0), 0)
    issue_gather(jnp.int32(0), 0)
    issue_idx(jnp.int32(1))

    def pair_body(j2, _):
        for u in range(2):
            j = 2 * j2 + u
            p = u
            q = 1 - u

            @pl.when(j >= 1)
            def _w():
                wait_scatter(q)

            @pl.when(j <= CPT - 2)
            def _og():
                wait_idx_and_offset(j + 1, q)
                issue_gather(j + 1, q)

            wait_gather(p)
            compute(p)
            issue_scatter(j, p)

            @pl.when(j <= CPT - 3)
            def _i():
                issue_idx(j + 2)
        return 0

    lax.fori_loop(0, CPT // 2, pair_body, 0)
    wait_scatter(1)
    plsc.subcore_barrier()

    o0 = c * N + r0
    pltpu.sync_copy(outx_sh.at[pl.ds(r0, SLAB)], outx_hbm.at[pl.ds(o0, SLAB)])

    @pl.when(s == TILES - 1)
    def _write_tail():
        t0 = TILES * SLAB
        ot = c * N + t0
        pltpu.sync_copy(outx_sh.at[pl.ds(t0, TAIL)],
                        outx_hbm.at[pl.ds(ot, TAIL)])


def _edge_phase_sc(featx_tbl, tblB, src, dst):
    mesh = plsc.VectorSubcoreMesh(core_axis_name="c", subcore_axis_name="s")
    f = pl.kernel(
        _sc_body,
        compiler_params=pltpu.CompilerParams(use_tc_tiling_on_sc=False),
        out_type=jax.ShapeDtypeStruct((2 * N, 80), jnp.float32),
        mesh=mesh,
        scratch_types=[
            pltpu.VMEM_SHARED((N, 80), jnp.float32),
            pltpu.VMEM((4, CHUNK), jnp.int32),
            pltpu.VMEM((4, CHUNK), jnp.int32),
            pltpu.VMEM((CHUNK,), jnp.int32),
            pltpu.VMEM((CHUNK,), jnp.int32),
            pltpu.VMEM((CHUNK, 80), jnp.float32),
            pltpu.VMEM((CHUNK, 80), jnp.float32),
            pltpu.VMEM((CHUNK, 16), jnp.float32),
            pltpu.VMEM((CHUNK, 16), jnp.float32),
            pltpu.SemaphoreType.DMA,
            pltpu.SemaphoreType.DMA,
            pltpu.SemaphoreType.DMA,
            pltpu.SemaphoreType.DMA,
            pltpu.SemaphoreType.DMA,
        ],
    )
    z80 = jnp.zeros((N, 80), jnp.float32)
    return f(featx_tbl, tblB, src, dst, z80)


def _build_alar(al, ar):
    idx = jnp.arange(HID)
    head = idx // DH
    A = jnp.zeros((HID, 2 * H), jnp.float32)
    A = A.at[idx, head].set(al.reshape(-1))
    A = A.at[idx, H + head].set(ar.reshape(-1))
    return A


def _expand_mat():
    idx = jnp.arange(HID)
    return (jnp.arange(H)[:, None] == (idx // DH)[None, :]).astype(jnp.float32)


def kernel(x, edge_index, W1, al1, ar1, b1, W2, al2, ar2, b2, ln_g, ln_b,
           Wc1, bc1, Wc2, bc2):
    src = edge_index[0]
    dst = edge_index[1]
    EXPAND = _expand_mat()

    ft1, tB1 = _stage1(x, W1, _build_alar(al1, ar1))
    outx1 = _edge_phase_sc(ft1, tB1, src, dst)
    h1, ft2, tB2 = _stage2(outx1, b1, W2, _build_alar(al2, ar2), EXPAND)
    outx2 = _edge_phase_sc(ft2, tB2, src, dst)
    return _stage3(outx2, h1, b2, ln_g, ln_b, Wc1, bc1, Wc2, bc2, EXPAND)
